# Initial kernel scaffold; baseline (speedup 1.0000x reference)
import jax
import jax.numpy as jnp
from jax.experimental import pallas as pl


def kernel(tokens, table):
    raise NotImplementedError("write your pallas kernel here")


# speedup vs baseline: 7.5257x; 7.5257x over previous
import functools
import math

import jax
import jax.numpy as jnp
from jax import lax
from jax.experimental import pallas as pl
from jax.experimental.pallas import tpu as pltpu
from jax.experimental.pallas import tpu_sc as plsc

_VOCAB = 100000
_EMB = 128
_SCALE = math.sqrt(float(_EMB))

_NC = 2
_NS = 16
_NW = _NC * _NS

_B = 4096 * 200
_B_PER_W = _B // _NW
_CHUNK = 128
_N_CHUNKS = _B_PER_W // _CHUNK
_NBUF = 4


def _scale_body(t_ref, o_ref):
    o_ref[...] = t_ref[...] * _SCALE


_scale_table = pl.pallas_call(
    _scale_body,
    grid=(100,),
    in_specs=[pl.BlockSpec((_VOCAB // 100, _EMB), lambda i: (i, 0))],
    out_specs=pl.BlockSpec((_VOCAB // 100, _EMB), lambda i: (i, 0)),
    out_shape=jax.ShapeDtypeStruct((_VOCAB, _EMB), jnp.float32),
)


_mesh = plsc.VectorSubcoreMesh(
    core_axis_name="c", subcore_axis_name="s", num_cores=_NC, num_subcores=_NS
)


@functools.partial(
    pl.kernel,
    mesh=_mesh,
    out_type=jax.ShapeDtypeStruct((_B, _EMB), jnp.float32),
    scratch_types=[
        pltpu.VMEM((_B_PER_W,), jnp.int32),
        pltpu.VMEM((_NBUF, _CHUNK, _EMB), jnp.float32),
        pltpu.SemaphoreType.DMA,
        pltpu.SemaphoreType.DMA,
        pltpu.SemaphoreType.DMA,
        pltpu.SemaphoreType.DMA,
        pltpu.SemaphoreType.DMA,
        pltpu.SemaphoreType.DMA,
        pltpu.SemaphoreType.DMA,
        pltpu.SemaphoreType.DMA,
    ],
)
def _sc_gather(tokens_hbm, table_hbm, out_hbm, idx_v, rows_v,
               g0, g1, g2, g3, s0, s1, s2, s3):
    gsems = [g0, g1, g2, g3]
    ssems = [s0, s1, s2, s3]
    wid = lax.axis_index("s") * _NC + lax.axis_index("c")
    base = wid * _B_PER_W

    pltpu.sync_copy(tokens_hbm.at[pl.ds(base, _B_PER_W)], idx_v)

    def gather_start(g, b):
        isl = idx_v.at[pl.ds(g * _CHUNK, _CHUNK)]
        pltpu.make_async_copy(table_hbm.at[isl], rows_v.at[b], gsems[b]).start()

    for b in range(_NBUF):
        gather_start(b, b)

    def body(i, carry):
        for b in range(_NBUF):
            g = i * _NBUF + b
            isl = idx_v.at[pl.ds(g * _CHUNK, _CHUNK)]
            pltpu.make_async_copy(table_hbm.at[isl], rows_v.at[b], gsems[b]).wait()
            dst = out_hbm.at[pl.ds(base + g * _CHUNK, _CHUNK)]
            scopy = pltpu.make_async_copy(rows_v.at[b], dst, ssems[b])
            scopy.start()
            scopy.wait()
            nxt = g + _NBUF

            @pl.when(nxt < _N_CHUNKS)
            def _():
                gather_start(nxt, b)
        return carry

    lax.fori_loop(0, _N_CHUNKS // _NBUF, body, 0)


def kernel(tokens, table):
    flat = tokens.reshape(-1).astype(jnp.int32)
    scaled = _scale_table(table)
    out = _sc_gather(flat, scaled)
    return out.reshape(tokens.shape + (_EMB,))
